# baseline (speedup 1.0000x reference)
import functools

import jax
import jax.numpy as jnp
from jax import lax
from jax.experimental import pallas as pl
from jax.experimental.pallas import tpu as pltpu
from jax.experimental.pallas import tpu_sc as plsc


def _loss1_sums(inp3, target, output):
    B, C, H, W = target.shape
    NB = 16

    def body(in_ref, t_ref, o_ref, sabs_ref, sin_ref, acc_abs, acc_in):
        b = pl.program_id(0)

        @pl.when(b == 0)
        def _():
            acc_abs[...] = jnp.zeros_like(acc_abs)
            acc_in[...] = jnp.zeros_like(acc_in)

        acc = acc_abs[...]
        acci = acc_in[...]
        for nb in range(NB):
            inb = in_ref[nb]
            for c in range(C):
                acc = acc + jnp.abs(
                    jnp.where(inb != 0.0, o_ref[nb, c], 0.0) - t_ref[nb, c]
                )
            acci = acci + inb
        acc_abs[...] = acc
        acc_in[...] = acci

        @pl.when(b == pl.num_programs(0) - 1)
        def _():
            sabs_ref[0, 0] = jnp.sum(acc_abs[...])
            sin_ref[0, 0] = jnp.sum(acc_in[...])

    return pl.pallas_call(
        body,
        grid=(B // NB,),
        in_specs=[
            pl.BlockSpec((NB, H, W), lambda b: (b, 0, 0)),
            pl.BlockSpec((NB, C, H, W), lambda b: (b, 0, 0, 0)),
            pl.BlockSpec((NB, C, H, W), lambda b: (b, 0, 0, 0)),
        ],
        out_specs=[
            pl.BlockSpec((1, 1), lambda b: (0, 0), memory_space=pltpu.SMEM),
            pl.BlockSpec((1, 1), lambda b: (0, 0), memory_space=pltpu.SMEM),
        ],
        out_shape=[jax.ShapeDtypeStruct((1, 1), jnp.float32)] * 2,
        scratch_shapes=[pltpu.VMEM((H, W), jnp.float32)] * 2,
    )(inp3, target, output)


def _sc_gather_diff(tflat, oflat, mr2, B, C, H, W, L):
    out_rows, out_cols = B * C, L
    slab = 8
    nwu = out_rows // slab
    lanes = 128
    per_row = out_cols // lanes
    idx_rows = slab * per_row
    mr_rows = 4
    try:
        info = plsc.get_sparse_core_info()
        nc, ns = info.num_cores, info.num_subcores
    except Exception:
        nc, ns = 2, 16
    mesh = plsc.VectorSubcoreMesh(
        core_axis_name="c", subcore_axis_name="s", num_cores=nc, num_subcores=ns
    )

    @functools.partial(
        pl.kernel,
        out_type=jax.ShapeDtypeStruct((out_rows, out_cols), jnp.float32),
        mesh=mesh,
        compiler_params=pltpu.CompilerParams(needs_layout_passes=False),
        scratch_types=[
            pltpu.VMEM((mr_rows * 2 * L,), jnp.int32),
            pltpu.VMEM((idx_rows, lanes), jnp.int32),
            pltpu.VMEM((slab, out_cols), jnp.float32),
            pltpu.VMEM((slab, out_cols), jnp.float32),
            pltpu.SemaphoreType.DMA,
        ],
    )
    def k(t_hbm, o_hbm, mr_hbm, out_hbm, mr_v, idx_v, tv, ov, sem):
        wid = lax.axis_index("s") * nc + lax.axis_index("c")

        @pl.when(wid < nwu)
        def _():
            bc0 = wid * slab
            b_first = bc0 // C
            mr_base = jnp.minimum(b_first, B - mr_rows)
            pltpu.sync_copy(
                mr_hbm.at[pl.ds(mr_base * (2 * L), mr_rows * 2 * L)], mr_v
            )
            for m in range(slab):
                bc = bc0 + m
                row0 = jnp.broadcast_to((bc // C - mr_base) * (2 * L), (16,))
                coef = jnp.broadcast_to(bc * (H * W), (16,))
                for i in range(out_cols // 16):
                    pos_r = row0 + lax.iota(jnp.int32, 16) * 2 + 32 * i
                    rv = plsc.load_gather(mr_v, [pos_r])
                    cv = plsc.load_gather(mr_v, [pos_r + 1])
                    idxv = coef + rv * W + cv
                    idx_v[m * per_row + i // 8, pl.ds((i % 8) * 16, 16)] = idxv
            copies = []
            for j in range(idx_rows):
                dst = (j // per_row, pl.ds((j % per_row) * lanes, lanes))
                copies.append(pltpu.async_copy(t_hbm.at[idx_v.at[j]], tv.at[dst], sem))
                copies.append(pltpu.async_copy(o_hbm.at[idx_v.at[j]], ov.at[dst], sem))
            for cp in copies:
                cp.wait()
            for j in range(slab):
                for i in range(out_cols // 16):
                    sl = pl.ds(i * 16, 16)
                    tv[j, sl] = tv[j, sl] - ov[j, sl]
            base = pl.multiple_of(wid * slab, 8)
            pltpu.sync_copy(tv, out_hbm.at[pl.ds(base, slab)])

    return k(tflat, oflat, mr2)


def _combine(g2, bmat, pI_row, sabs, sin, C):
    BC, KL = g2.shape
    S, D = bmat.shape
    K = KL // D

    def body(g_ref, b_ref, pI_ref, sabs_ref, sin_ref, loss_ref, l1_ref, l2_ref):
        bm = b_ref[...]
        pI = pI_ref[...]
        E = jnp.zeros((BC, S), jnp.float32)
        for k in range(K):
            gk = g_ref[:, k * D:(k + 1) * D]
            Mk = lax.dot_general(
                gk, bm, (((1,), (1,)), ((), ())),
                preferred_element_type=jnp.float32,
            )
            E = E + Mk * (pI == k).astype(jnp.float32)
        s2 = jnp.sum(jnp.abs(E))
        l1 = sabs_ref[0, 0] / (C * sin_ref[0, 0])
        l2 = s2 / (BC * S)
        l1_ref[0, 0] = l1
        l2_ref[0, 0] = l2
        loss_ref[0, 0] = l1 + l2

    return pl.pallas_call(
        body,
        in_specs=[
            pl.BlockSpec(memory_space=pltpu.VMEM),
            pl.BlockSpec(memory_space=pltpu.VMEM),
            pl.BlockSpec(memory_space=pltpu.VMEM),
            pl.BlockSpec(memory_space=pltpu.SMEM),
            pl.BlockSpec(memory_space=pltpu.SMEM),
        ],
        out_specs=[
            pl.BlockSpec(memory_space=pltpu.SMEM),
            pl.BlockSpec(memory_space=pltpu.SMEM),
            pl.BlockSpec(memory_space=pltpu.SMEM),
        ],
        out_shape=[jax.ShapeDtypeStruct((1, 1), jnp.float32)] * 3,
    )(g2, bmat, pI_row, sabs, sin)


def kernel(input, mapRecord, target, output, patchIndex, basis):
    B, C, H, W = output.shape
    L = mapRecord.shape[1]
    S = basis.shape[0]

    inp3 = input.reshape(B, H, W)
    sabs, sin = _loss1_sums(inp3, target, output)

    mr2 = mapRecord.reshape(-1)
    g2 = _sc_gather_diff(
        target.reshape(-1), output.reshape(-1), mr2, B, C, H, W, L
    )

    bmat = basis.reshape(S, -1)
    pI_row = patchIndex.reshape(1, S)
    loss, l1, l2 = _combine(g2, bmat, pI_row, sabs, sin, C)
    return loss.reshape(()), l1.reshape(()), l2.reshape(())


# speedup vs baseline: 1.3754x; 1.3754x over previous
import functools

import jax
import jax.numpy as jnp
from jax import lax
from jax.experimental import pallas as pl
from jax.experimental.pallas import tpu as pltpu
from jax.experimental.pallas import tpu_sc as plsc


def _loss1_sums(inp3, target, output):
    B, C, H, W = target.shape
    NB = 16

    def body(in_ref, t_ref, o_ref, sabs_ref, sin_ref, acc_abs, acc_in):
        b = pl.program_id(0)

        @pl.when(b == 0)
        def _():
            acc_abs[...] = jnp.zeros_like(acc_abs)
            acc_in[...] = jnp.zeros_like(acc_in)

        acc = acc_abs[...]
        acci = acc_in[...]
        for nb in range(NB):
            inb = in_ref[nb]
            for c in range(C):
                acc = acc + jnp.abs(
                    jnp.where(inb != 0.0, o_ref[nb, c], 0.0) - t_ref[nb, c]
                )
            acci = acci + inb
        acc_abs[...] = acc
        acc_in[...] = acci

        @pl.when(b == pl.num_programs(0) - 1)
        def _():
            sabs_ref[0, 0] = jnp.sum(acc_abs[...])
            sin_ref[0, 0] = jnp.sum(acc_in[...])

    return pl.pallas_call(
        body,
        grid=(B // NB,),
        in_specs=[
            pl.BlockSpec((NB, H, W), lambda b: (b, 0, 0)),
            pl.BlockSpec((NB, C, H, W), lambda b: (b, 0, 0, 0)),
            pl.BlockSpec((NB, C, H, W), lambda b: (b, 0, 0, 0)),
        ],
        out_specs=[
            pl.BlockSpec((1, 1), lambda b: (0, 0), memory_space=pltpu.SMEM),
            pl.BlockSpec((1, 1), lambda b: (0, 0), memory_space=pltpu.SMEM),
        ],
        out_shape=[jax.ShapeDtypeStruct((1, 1), jnp.float32)] * 2,
        scratch_shapes=[pltpu.VMEM((H, W), jnp.float32)] * 2,
    )(inp3, target, output)


def _sc_gather_diff(tflat, oflat, off1d, B, C, H, W, L):
    out_rows, out_cols = B * C, L
    slab = 8
    nwu = out_rows // slab
    lanes = 128
    per_row = out_cols // lanes
    idx_rows = slab * per_row
    off_rows = 4
    try:
        info = plsc.get_sparse_core_info()
        nc, ns = info.num_cores, info.num_subcores
    except Exception:
        nc, ns = 2, 16
    mesh = plsc.VectorSubcoreMesh(
        core_axis_name="c", subcore_axis_name="s", num_cores=nc, num_subcores=ns
    )

    @functools.partial(
        pl.kernel,
        out_type=jax.ShapeDtypeStruct((out_rows, 2 * out_cols), jnp.float32),
        mesh=mesh,
        scratch_types=[
            pltpu.VMEM((off_rows * L,), jnp.int32),
            pltpu.VMEM((idx_rows, lanes), jnp.int32),
            pltpu.VMEM((slab, out_cols), jnp.float32),
            pltpu.VMEM((slab, out_cols), jnp.float32),
            pltpu.SemaphoreType.DMA,
        ],
    )
    def k(t_hbm, o_hbm, off_hbm, out_hbm, off_v, idx_v, tv, ov, sem):
        wid = lax.axis_index("s") * nc + lax.axis_index("c")

        @pl.when(wid < nwu)
        def _():
            bc0 = wid * slab
            b0 = jnp.minimum(bc0 // C, B - off_rows)
            src0 = pl.multiple_of(b0 * L, 8)
            pltpu.sync_copy(off_hbm.at[pl.ds(src0, off_rows * L)], off_v)
            for m in range(slab):
                bc = bc0 + m
                basev = jnp.broadcast_to(bc * (H * W), (16,))
                row0 = (bc // C - b0) * L
                for i in range(out_cols // 16):
                    src = pl.multiple_of(row0 + i * 16, 8)
                    ov16 = off_v[pl.ds(src, 16)]
                    idx_v[m * per_row + i // 8, pl.ds((i % 8) * 16, 16)] = (
                        ov16 + basev
                    )
            copies = []
            for j in range(idx_rows):
                dst = (j // per_row, pl.ds((j % per_row) * lanes, lanes))
                copies.append(pltpu.async_copy(t_hbm.at[idx_v.at[j]], tv.at[dst], sem))
                copies.append(pltpu.async_copy(o_hbm.at[idx_v.at[j]], ov.at[dst], sem))
            for cp in copies:
                cp.wait()
            base = pl.multiple_of(wid * slab, 8)
            pltpu.sync_copy(tv, out_hbm.at[pl.ds(base, slab), pl.ds(0, out_cols)])
            pltpu.sync_copy(
                ov, out_hbm.at[pl.ds(base, slab), pl.ds(out_cols, out_cols)]
            )

    return k(tflat, oflat, off1d)


def _combine(gto, bT, pI_row, sabs, sin, C):
    BC, KL2 = gto.shape
    KL = KL2 // 2
    D, S = bT.shape
    K = KL // D

    def body(g_ref, b_ref, pI_ref, sabs_ref, sin_ref, loss_ref, l1_ref, l2_ref):
        bm = b_ref[...]
        pI = pI_ref[...]
        diff = g_ref[:, :KL] - g_ref[:, KL:]
        E = jnp.zeros((BC, S), jnp.float32)
        for k in range(K):
            gk = diff[:, k * D:(k + 1) * D]
            Mk = jnp.dot(gk, bm, preferred_element_type=jnp.float32)
            E = E + Mk * (pI == k).astype(jnp.float32)
        s2 = jnp.sum(jnp.abs(E))
        l1 = sabs_ref[0, 0] / (C * sin_ref[0, 0])
        l2 = s2 / (BC * S)
        l1_ref[0, 0] = l1
        l2_ref[0, 0] = l2
        loss_ref[0, 0] = l1 + l2

    return pl.pallas_call(
        body,
        in_specs=[
            pl.BlockSpec(memory_space=pltpu.VMEM),
            pl.BlockSpec(memory_space=pltpu.VMEM),
            pl.BlockSpec(memory_space=pltpu.VMEM),
            pl.BlockSpec(memory_space=pltpu.SMEM),
            pl.BlockSpec(memory_space=pltpu.SMEM),
        ],
        out_specs=[
            pl.BlockSpec(memory_space=pltpu.SMEM),
            pl.BlockSpec(memory_space=pltpu.SMEM),
            pl.BlockSpec(memory_space=pltpu.SMEM),
        ],
        out_shape=[jax.ShapeDtypeStruct((1, 1), jnp.float32)] * 3,
    )(gto, bT, pI_row, sabs, sin)


def kernel(input, mapRecord, target, output, patchIndex, basis):
    B, C, H, W = output.shape
    L = mapRecord.shape[1]
    S = basis.shape[0]

    inp3 = input.reshape(B, H, W)
    sabs, sin = _loss1_sums(inp3, target, output)

    off1d = (mapRecord[:, :, 0] * W + mapRecord[:, :, 1]).reshape(-1)
    gto = _sc_gather_diff(
        target.reshape(-1), output.reshape(-1), off1d, B, C, H, W, L
    )

    bT = basis.reshape(S, -1).T
    pI_row = patchIndex.reshape(1, S)
    loss, l1, l2 = _combine(gto, bT, pI_row, sabs, sin, C)
    return loss.reshape(()), l1.reshape(()), l2.reshape(())
